# Initial kernel scaffold; baseline (speedup 1.0000x reference)
import functools
import math

import jax
import jax.numpy as jnp
from jax import lax
from jax.experimental import pallas as pl
from jax.experimental.pallas import tpu as pltpu
from jax.experimental.pallas import tpu_sc as plsc

NC = 2
NS = 16
NW = NC * NS

CH = 128
NBUF = 5


def _make_scale_body(scale):
    def body(x_ref, o_ref):
        o_ref[...] = x_ref[...] * scale
    return body


def _prescale(table, scale):
    V, D = table.shape
    blk = 1000
    return pl.pallas_call(
        _make_scale_body(scale),
        grid=(pl.cdiv(V, blk),),
        in_specs=[pl.BlockSpec((blk, D), lambda i: (i, 0))],
        out_specs=pl.BlockSpec((blk, D), lambda i: (i, 0)),
        out_shape=jax.ShapeDtypeStruct((V, D), table.dtype),
    )(table)


def _sc_gather(table, idx2d):
    T, _ = idx2d.shape
    V, D = table.shape
    nchunk = T // NW
    nouter = -(-nchunk // NBUF)
    nprime = min(NBUF, nchunk)
    mesh = plsc.VectorSubcoreMesh(core_axis_name="c", subcore_axis_name="s")

    @functools.partial(
        pl.kernel,
        out_type=jax.ShapeDtypeStruct((T * CH, D), jnp.float32),
        mesh=mesh,
        scratch_types=[
            pltpu.VMEM((nchunk, CH), jnp.int32),
            pltpu.VMEM((NBUF, CH, D), jnp.float32),
            pltpu.SemaphoreType.DMA((NBUF,)),
        ],
    )
    def run(table_hbm, idx_hbm, out_hbm, idx_v, rows_v, sems):
        wid = lax.axis_index("s") * NC + lax.axis_index("c")
        cbase = wid * nchunk
        rbase = cbase * CH
        pltpu.sync_copy(idx_hbm.at[pl.ds(cbase, nchunk)], idx_v)
        for b in range(nprime):
            pltpu.async_copy(table_hbm.at[idx_v.at[b]], rows_v.at[b], sems.at[b])

        def outer(g, carry):
            for b in range(NBUF):
                j = g * NBUF + b

                @pl.when(j < nchunk)
                def _():
                    pltpu.make_async_copy(
                        table_hbm.at[idx_v.at[b]], rows_v.at[b], sems.at[b]
                    ).wait()
                    pltpu.sync_copy(
                        rows_v.at[b], out_hbm.at[pl.ds(rbase + j * CH, CH)]
                    )
                    jn = j + NBUF

                    @pl.when(jn < nchunk)
                    def _():
                        pltpu.async_copy(
                            table_hbm.at[idx_v.at[jn]], rows_v.at[b], sems.at[b]
                        )
            return carry

        lax.fori_loop(0, nouter, outer, 0)

    return run(table, idx2d)


def kernel(tokens, embedding):
    B, S = tokens.shape
    V, D = embedding.shape
    N = B * S
    scaled = _prescale(embedding, math.sqrt(D))
    idx = tokens.reshape(N).astype(jnp.int32)
    span = NW * CH
    NP = -(-N // span) * span
    if NP != N:
        idx = jnp.concatenate([idx, jnp.zeros((NP - N,), jnp.int32)])
    out = _sc_gather(scaled, idx.reshape(NP // CH, CH))
    if NP != N:
        out = out[:N]
    return out.reshape(B, S, D)


# speedup vs baseline: 2.3641x; 2.3641x over previous
import functools
import math

import jax
import jax.numpy as jnp
from jax import lax
from jax.experimental import pallas as pl
from jax.experimental.pallas import tpu as pltpu
from jax.experimental.pallas import tpu_sc as plsc

NC = 2
NS = 16
NW = NC * NS

CH = 128
NBUF = 5


def _make_scale_body(scale):
    def body(x_ref, o_ref):
        o_ref[...] = x_ref[...] * scale
    return body


def _prescale(table, scale):
    V, D = table.shape
    blk = 1000
    return pl.pallas_call(
        _make_scale_body(scale),
        grid=(pl.cdiv(V, blk),),
        in_specs=[pl.BlockSpec((blk, D), lambda i: (i, 0))],
        out_specs=pl.BlockSpec((blk, D), lambda i: (i, 0)),
        out_shape=jax.ShapeDtypeStruct((V, D), table.dtype),
    )(table)


def _sc_gather(table, idx):
    NP, = idx.shape
    V, D = table.shape
    npw = NP // NW
    nchunk = npw // CH
    nouter = -(-nchunk // NBUF)
    nprime = min(NBUF, nchunk)
    mesh = plsc.VectorSubcoreMesh(core_axis_name="c", subcore_axis_name="s")

    @functools.partial(
        pl.kernel,
        out_type=jax.ShapeDtypeStruct((NP, D), jnp.float32),
        mesh=mesh,
        scratch_types=[
            pltpu.VMEM((npw,), jnp.int32),
            pltpu.VMEM((NBUF, CH, D), jnp.float32),
            pltpu.SemaphoreType.DMA((NBUF,)),
        ],
    )
    def run(table_hbm, idx_hbm, out_hbm, idx_v, rows_v, sems):
        wid = lax.axis_index("s") * NC + lax.axis_index("c")
        rbase = wid * npw
        pltpu.sync_copy(idx_hbm.at[pl.ds(rbase, npw)], idx_v)
        for b in range(nprime):
            pltpu.async_copy(
                table_hbm.at[idx_v.at[pl.ds(b * CH, CH)]], rows_v.at[b], sems.at[b]
            )

        def outer(g, carry):
            for b in range(NBUF):
                j = g * NBUF + b

                @pl.when(j < nchunk)
                def _():
                    pltpu.make_async_copy(
                        table_hbm.at[idx_v.at[pl.ds(0, CH)]], rows_v.at[b], sems.at[b]
                    ).wait()
                    pltpu.sync_copy(
                        rows_v.at[b], out_hbm.at[pl.ds(rbase + j * CH, CH)]
                    )
                    jn = j + NBUF

                    @pl.when(jn < nchunk)
                    def _():
                        pltpu.async_copy(
                            table_hbm.at[idx_v.at[pl.ds(jn * CH, CH)]],
                            rows_v.at[b], sems.at[b]
                        )
            return carry

        lax.fori_loop(0, nouter, outer, 0)

    return run(table, idx)


def kernel(tokens, embedding):
    B, S = tokens.shape
    V, D = embedding.shape
    N = B * S
    scaled = _prescale(embedding, math.sqrt(D))
    idx = tokens.reshape(N).astype(jnp.int32)
    span = NW * CH
    NP = -(-N // span) * span
    if NP != N:
        idx = jnp.concatenate([idx, jnp.zeros((NP - N,), jnp.int32)])
    out = _sc_gather(scaled, idx)
    if NP != N:
        out = out[:N]
    return out.reshape(B, S, D)


# speedup vs baseline: 3.6526x; 1.5450x over previous
import functools
import math

import jax
import jax.numpy as jnp
from jax import lax
from jax.experimental import pallas as pl
from jax.experimental.pallas import tpu as pltpu
from jax.experimental.pallas import tpu_sc as plsc

NC = 2
NS = 16
NW = NC * NS

CH = 128
NBUF = 8


def _make_scale_body(scale):
    def body(x_ref, o_ref):
        o_ref[...] = x_ref[...] * scale
    return body


def _prescale(table, scale):
    V, D = table.shape
    blk = 1000
    return pl.pallas_call(
        _make_scale_body(scale),
        grid=(pl.cdiv(V, blk),),
        in_specs=[pl.BlockSpec((blk, D), lambda i: (i, 0))],
        out_specs=pl.BlockSpec((blk, D), lambda i: (i, 0)),
        out_shape=jax.ShapeDtypeStruct((V, D), table.dtype),
    )(table)


def _sc_gather_seq(table, tok):
    B, S = tok.shape
    V, D = table.shape
    nsq = B // NW
    nouter = -(-nsq // NBUF)
    nprime = min(NBUF, nsq)
    mesh = plsc.VectorSubcoreMesh(core_axis_name="c", subcore_axis_name="s")

    @functools.partial(
        pl.kernel,
        out_type=jax.ShapeDtypeStruct((B, S, D), jnp.float32),
        mesh=mesh,
        scratch_types=[
            pltpu.VMEM((nsq, S), jnp.int32),
            pltpu.VMEM((NBUF, S, D), jnp.float32),
            pltpu.SemaphoreType.DMA((NBUF,)),
        ],
    )
    def run(table_hbm, tok_hbm, out_hbm, idx_v, rows_v, sems):
        wid = lax.axis_index("s") * NC + lax.axis_index("c")
        sbase = wid * nsq
        pltpu.sync_copy(tok_hbm.at[pl.ds(sbase, nsq)], idx_v)
        for b in range(nprime):
            pltpu.async_copy(
                table_hbm.at[idx_v.at[b]], rows_v.at[b], sems.at[b]
            )

        def outer(g, carry):
            for b in range(NBUF):
                q = g * NBUF + b

                @pl.when(q < nsq)
                def _():
                    pltpu.make_async_copy(
                        table_hbm.at[idx_v.at[0]], rows_v.at[b], sems.at[b]
                    ).wait()
                    pltpu.sync_copy(rows_v.at[b], out_hbm.at[sbase + q])
                    qn = q + NBUF

                    @pl.when(qn < nsq)
                    def _():
                        pltpu.async_copy(
                            table_hbm.at[idx_v.at[qn]], rows_v.at[b], sems.at[b]
                        )
            return carry

        lax.fori_loop(0, nouter, outer, 0)

    return run(table, tok)


def _sc_gather_flat(table, idx):
    NP, = idx.shape
    V, D = table.shape
    npw = NP // NW
    nchunk = npw // CH
    nouter = -(-nchunk // NBUF)
    nprime = min(NBUF, nchunk)
    mesh = plsc.VectorSubcoreMesh(core_axis_name="c", subcore_axis_name="s")

    @functools.partial(
        pl.kernel,
        out_type=jax.ShapeDtypeStruct((NP, D), jnp.float32),
        mesh=mesh,
        scratch_types=[
            pltpu.VMEM((npw,), jnp.int32),
            pltpu.VMEM((NBUF, CH, D), jnp.float32),
            pltpu.SemaphoreType.DMA((NBUF,)),
        ],
    )
    def run(table_hbm, idx_hbm, out_hbm, idx_v, rows_v, sems):
        wid = lax.axis_index("s") * NC + lax.axis_index("c")
        rbase = wid * npw
        pltpu.sync_copy(idx_hbm.at[pl.ds(rbase, npw)], idx_v)
        for b in range(nprime):
            pltpu.async_copy(
                table_hbm.at[idx_v.at[pl.ds(b * CH, CH)]], rows_v.at[b], sems.at[b]
            )

        def outer(g, carry):
            for b in range(NBUF):
                j = g * NBUF + b

                @pl.when(j < nchunk)
                def _():
                    pltpu.make_async_copy(
                        table_hbm.at[idx_v.at[pl.ds(0, CH)]], rows_v.at[b], sems.at[b]
                    ).wait()
                    pltpu.sync_copy(
                        rows_v.at[b], out_hbm.at[pl.ds(rbase + j * CH, CH)]
                    )
                    jn = j + NBUF

                    @pl.when(jn < nchunk)
                    def _():
                        pltpu.async_copy(
                            table_hbm.at[idx_v.at[pl.ds(jn * CH, CH)]],
                            rows_v.at[b], sems.at[b]
                        )
            return carry

        lax.fori_loop(0, nouter, outer, 0)

    return run(table, idx)


def kernel(tokens, embedding):
    B, S = tokens.shape
    V, D = embedding.shape
    N = B * S
    scaled = _prescale(embedding, math.sqrt(D))
    if B % NW == 0 and S <= 128:
        return _sc_gather_seq(scaled, tokens.astype(jnp.int32))
    idx = tokens.reshape(N).astype(jnp.int32)
    span = NW * CH
    NP = -(-N // span) * span
    if NP != N:
        idx = jnp.concatenate([idx, jnp.zeros((NP - N,), jnp.int32)])
    out = _sc_gather_flat(scaled, idx)
    if NP != N:
        out = out[:N]
    return out.reshape(B, S, D)
